# baseline (speedup 1.0000x reference)
import functools

import jax
import jax.numpy as jnp
from jax import lax
from jax.experimental import pallas as pl
from jax.experimental.pallas import tpu as pltpu
from jax.experimental.pallas import tpu_sc as plsc

V = 100000
T = 64
D = 128
B = 1024
NP = 4

VB = 2048
NB = (V + VB - 1) // VB

_INTERPRET = False


def _sc_gather_combine(idx_a, idx_b, idx_c, turns):
    info = plsc.get_sparse_core_info()
    nw = info.num_cores * info.num_subcores
    bw = B // nw

    mesh = plsc.VectorSubcoreMesh(core_axis_name="c", subcore_axis_name="s")

    @functools.partial(
        pl.kernel,
        out_type=jax.ShapeDtypeStruct((B, T), jnp.float32),
        mesh=mesh,
        compiler_params=pltpu.CompilerParams(use_tc_tiling_on_sc=False),
        scratch_types=[
            pltpu.VMEM((bw,), jnp.int32),
            pltpu.VMEM((bw,), jnp.int32),
            pltpu.VMEM((bw,), jnp.int32),
            pltpu.VMEM((bw, T), jnp.float32),
            pltpu.VMEM((bw, T), jnp.float32),
            pltpu.VMEM((bw, T), jnp.float32),
            pltpu.VMEM((bw, T), jnp.float32),
            pltpu.SemaphoreType.DMA,
            pltpu.SemaphoreType.DMA,
            pltpu.SemaphoreType.DMA,
        ],
    )
    def sc_kernel(ia_hbm, ib_hbm, ic_hbm, turns_hbm, out_hbm,
                  ia_v, ib_v, ic_v, ra_v, rb_v, rc_v, out_v,
                  sem_a, sem_b, sem_c):
        wid = lax.axis_index("s") * info.num_cores + lax.axis_index("c")
        base = wid * bw
        pltpu.sync_copy(ia_hbm.at[pl.ds(base, bw)], ia_v)
        pltpu.sync_copy(ib_hbm.at[pl.ds(base, bw)], ib_v)
        pltpu.sync_copy(ic_hbm.at[pl.ds(base, bw)], ic_v)
        da = pltpu.async_copy(turns_hbm.at[ia_v], ra_v, sem_a)
        db = pltpu.async_copy(turns_hbm.at[ib_v], rb_v, sem_b)
        dc = pltpu.async_copy(turns_hbm.at[ic_v], rc_v, sem_c)
        da.wait()
        db.wait()
        dc.wait()
        for i in range(bw):
            for c in range(T // 16):
                sl = pl.ds(c * 16, 16)
                out_v[i, sl] = ra_v[i, sl] - rb_v[i, sl] + rc_v[i, sl]
        pltpu.sync_copy(out_v, out_hbm.at[pl.ds(base, bw)])

    return sc_kernel(idx_a, idx_b, idx_c, turns)


def _knn_body(res_ref, turns_ref, pc_ref, dist_ref, idx_ref, emb_ref,
              minv_ref):
    pid = pl.program_id(0)
    res = res_ref[...]
    tb = turns_ref[...]

    hi = jax.lax.Precision.HIGHEST
    qk = lax.dot_general(res, tb, (((1,), (1,)), ((), ())),
                         preferred_element_type=jnp.float32)
    k2 = lax.dot_general(jnp.ones((1, T), jnp.float32), tb * tb,
                         (((1,), (1,)), ((), ())),
                         preferred_element_type=jnp.float32,
                         precision=hi)
    d2 = k2 - 2.0 * qk

    def _update(d2v):
        bmin = jnp.min(d2v, axis=1, keepdims=True)
        bidx = jnp.argmin(d2v, axis=1, keepdims=True).astype(jnp.int32)
        gidx = bidx + pid * VB

        @pl.when(pid == 0)
        def _():
            minv_ref[...] = bmin
            idx_ref[...] = gidx

        @pl.when(pid > 0)
        def _():
            prev = minv_ref[...]
            better = bmin < prev
            minv_ref[...] = jnp.where(better, bmin, prev)
            idx_ref[...] = jnp.where(better, gidx, idx_ref[...])

    @pl.when(pid < NB - 1)
    def _():
        _update(d2)

    @pl.when(pid == NB - 1)
    def _():
        col = lax.broadcasted_iota(jnp.int32, (1, VB), 1) + pid * VB
        _update(jnp.where(col < V, d2, jnp.float32(1e30)))

        q2 = jnp.sum(res * res, axis=1, keepdims=True)
        dist_ref[...] = jnp.sqrt(jnp.maximum(minv_ref[...] + q2, 0.0))

        r2 = res * res
        powers = jnp.concatenate(
            [jnp.ones_like(res), res, r2, r2 * res], axis=1)
        emb_ref[...] = lax.dot_general(
            powers, pc_ref[...], (((1,), (0,)), ((), ())),
            preferred_element_type=jnp.float32, precision=hi)


def _tc_knn(result, turns, pc2):
    return pl.pallas_call(
        _knn_body,
        grid=(NB,),
        in_specs=[
            pl.BlockSpec((B, T), lambda i: (0, 0)),
            pl.BlockSpec((VB, T), lambda i: (i, 0)),
            pl.BlockSpec((NP * T, D), lambda i: (0, 0)),
        ],
        out_specs=[
            pl.BlockSpec((B, 1), lambda i: (0, 0)),
            pl.BlockSpec((B, 1), lambda i: (0, 0)),
            pl.BlockSpec((B, D), lambda i: (0, 0)),
        ],
        out_shape=[
            jax.ShapeDtypeStruct((B, 1), jnp.float32),
            jax.ShapeDtypeStruct((B, 1), jnp.int32),
            jax.ShapeDtypeStruct((B, D), jnp.float32),
        ],
        scratch_shapes=[pltpu.VMEM((B, 1), jnp.float32)],
        interpret=_INTERPRET,
    )(result, turns, pc2)


def kernel(idx_a, idx_b, idx_c, turns, poly_coeffs):
    result = _sc_gather_combine(idx_a, idx_b, idx_c, turns)
    pc2 = jnp.transpose(poly_coeffs, (1, 0, 2)).reshape(NP * T, D)
    dist, idx, emb = _tc_knn(result, turns, pc2)
    return (result, idx.reshape(B), dist.reshape(B), emb)
